# Initial kernel scaffold; baseline (speedup 1.0000x reference)
import jax
import jax.numpy as jnp
from jax.experimental import pallas as pl


def kernel(inputs):
    raise NotImplementedError("write your pallas kernel here")


# speedup vs baseline: 1.8074x; 1.8074x over previous
import functools

import jax
import jax.numpy as jnp
from jax import lax
from jax.experimental import pallas as pl
from jax.experimental.pallas import tpu as pltpu
from jax.experimental.pallas import tpu_sc as plsc

_B, _S, _F = 2, 4096, 4096
_NSEL = 16
_CSTRIDE = 256
_ROWS = _B * _S
_L = 16


def _make_sc_kernel():
    info = plsc.get_sparse_core_info()
    num_workers = info.num_cores * info.num_subcores
    rows_per_w = _ROWS // num_workers
    mesh = plsc.VectorSubcoreMesh(core_axis_name="c", subcore_axis_name="s")

    @functools.partial(
        pl.kernel,
        mesh=mesh,
        out_type=jax.ShapeDtypeStruct((_ROWS, _NSEL), jnp.float32),
        compiler_params=pltpu.CompilerParams(needs_layout_passes=False),
        scratch_types=[
            pltpu.VMEM((rows_per_w, 128), jnp.float32),
            pltpu.VMEM((rows_per_w, _NSEL), jnp.float32),
        ],
    )
    def sc_kernel(x_hbm, out_hbm, buf, out_v):
        wid = lax.axis_index("s") * info.num_cores + lax.axis_index("c")
        base = wid * rows_per_w
        lanes = lax.iota(jnp.int32, _L)
        zeros = jnp.zeros((_L,), jnp.int32)
        for j in range(_NSEL):
            pltpu.sync_copy(
                x_hbm.at[pl.ds(base, rows_per_w), pl.ds(j * _CSTRIDE, 128)],
                buf,
            )
            cols_j = jnp.full((_L,), j, jnp.int32)
            for k in range(rows_per_w // _L):
                rows = k * _L + lanes
                vals = plsc.load_gather(buf, [rows, zeros])
                plsc.store_scatter(out_v, [rows, cols_j], vals)
        pltpu.sync_copy(out_v, out_hbm.at[pl.ds(base, rows_per_w), :])

    return sc_kernel


_sc_kernel = _make_sc_kernel()


def kernel(inputs):
    x2d = inputs.reshape(_ROWS, _F)
    out = _sc_kernel(x2d)
    return out.reshape(_B, _S, _NSEL)


# speedup vs baseline: 2.0626x; 1.1412x over previous
import functools

import jax
import jax.numpy as jnp
from jax import lax
from jax.experimental import pallas as pl
from jax.experimental.pallas import tpu as pltpu
from jax.experimental.pallas import tpu_sc as plsc

_B, _S, _F = 2, 4096, 4096
_NSEL = 16
_CSTRIDE = 256
_ROWS = _B * _S
_L = 16
_NBUF = 2


def _make_sc_kernel():
    info = plsc.get_sparse_core_info()
    num_workers = info.num_cores * info.num_subcores
    rows_per_w = _ROWS // num_workers
    mesh = plsc.VectorSubcoreMesh(core_axis_name="c", subcore_axis_name="s")

    @functools.partial(
        pl.kernel,
        mesh=mesh,
        out_type=jax.ShapeDtypeStruct((_ROWS, _NSEL), jnp.float32),
        compiler_params=pltpu.CompilerParams(needs_layout_passes=False),
        scratch_types=[
            pltpu.VMEM((_NBUF, rows_per_w, 128), jnp.float32),
            pltpu.VMEM((rows_per_w, _NSEL), jnp.float32),
        ]
        + [pltpu.SemaphoreType.DMA] * _NBUF,
    )
    def sc_kernel(x_hbm, out_hbm, bufs, out_v, *sems):
        wid = lax.axis_index("s") * info.num_cores + lax.axis_index("c")
        base = wid * rows_per_w
        lanes = lax.iota(jnp.int32, _L)
        zeros = jnp.zeros((_L,), jnp.int32)

        def start(j):
            return pltpu.async_copy(
                x_hbm.at[pl.ds(base, rows_per_w), pl.ds(j * _CSTRIDE, 128)],
                bufs.at[j % _NBUF],
                sems[j % _NBUF],
            )

        handles = {}
        for j in range(_NBUF):
            handles[j] = start(j)
        for j in range(_NSEL):
            handles.pop(j).wait()
            cols_j = jnp.full((_L,), j, jnp.int32)
            buf = bufs.at[j % _NBUF]
            for k in range(rows_per_w // _L):
                rows = k * _L + lanes
                vals = plsc.load_gather(buf, [rows, zeros])
                plsc.store_scatter(out_v, [rows, cols_j], vals)
            if j + _NBUF < _NSEL:
                handles[j + _NBUF] = start(j + _NBUF)
        pltpu.sync_copy(out_v, out_hbm.at[pl.ds(base, rows_per_w), :])

    return sc_kernel


_sc_kernel = _make_sc_kernel()


def kernel(inputs):
    x2d = inputs.reshape(_ROWS, _F)
    out = _sc_kernel(x2d)
    return out.reshape(_B, _S, _NSEL)


# speedup vs baseline: 2.5036x; 1.2138x over previous
import functools

import jax
import jax.numpy as jnp
from jax import lax
from jax.experimental import pallas as pl
from jax.experimental.pallas import tpu as pltpu
from jax.experimental.pallas import tpu_sc as plsc

_B, _S, _F = 2, 4096, 4096
_NSEL = 16
_CSTRIDE = 256
_ROWS = _B * _S
_L = 16
_NBUF = 2
_SC_ROWS = 4096
_TC_BLK = 256


def _make_sc_kernel(n_rows):
    info = plsc.get_sparse_core_info()
    num_workers = info.num_cores * info.num_subcores
    rows_per_w = n_rows // num_workers
    mesh = plsc.VectorSubcoreMesh(core_axis_name="c", subcore_axis_name="s")

    @functools.partial(
        pl.kernel,
        mesh=mesh,
        out_type=jax.ShapeDtypeStruct((n_rows, _NSEL), jnp.float32),
        compiler_params=pltpu.CompilerParams(needs_layout_passes=False),
        scratch_types=[
            pltpu.VMEM((_NBUF, rows_per_w, 128), jnp.float32),
            pltpu.VMEM((rows_per_w, _NSEL), jnp.float32),
        ]
        + [pltpu.SemaphoreType.DMA] * _NBUF,
    )
    def sc_kernel(x_hbm, out_hbm, bufs, out_v, *sems):
        wid = lax.axis_index("s") * info.num_cores + lax.axis_index("c")
        base = wid * rows_per_w
        lanes = lax.iota(jnp.int32, _L)
        zeros = jnp.zeros((_L,), jnp.int32)

        def start(j):
            return pltpu.async_copy(
                x_hbm.at[pl.ds(base, rows_per_w), pl.ds(j * _CSTRIDE, 128)],
                bufs.at[j % _NBUF],
                sems[j % _NBUF],
            )

        handles = {}
        for j in range(_NBUF):
            handles[j] = start(j)
        for j in range(_NSEL):
            handles.pop(j).wait()
            cols_j = jnp.full((_L,), j, jnp.int32)
            buf = bufs.at[j % _NBUF]
            for k in range(rows_per_w // _L):
                rows = k * _L + lanes
                vals = plsc.load_gather(buf, [rows, zeros])
                plsc.store_scatter(out_v, [rows, cols_j], vals)
            if j + _NBUF < _NSEL:
                handles[j + _NBUF] = start(j + _NBUF)
        pltpu.sync_copy(out_v, out_hbm.at[pl.ds(base, rows_per_w), :])

    return sc_kernel


def _tc_body(*refs):
    in_refs, out_ref = refs[:-1], refs[-1]
    out_ref[...] = jnp.concatenate([r[...][:, :1] for r in in_refs], axis=1)


def _make_tc_kernel(row_off, n_rows):
    grid = (n_rows // _TC_BLK,)
    in_specs = [
        pl.BlockSpec(
            (_TC_BLK, 128),
            functools.partial(
                lambda j, r: (row_off // _TC_BLK + r, 2 * j), j
            ),
        )
        for j in range(_NSEL)
    ]
    out_specs = pl.BlockSpec((_TC_BLK, _NSEL), lambda r: (r, 0))
    return pl.pallas_call(
        _tc_body,
        grid=grid,
        in_specs=in_specs,
        out_specs=out_specs,
        out_shape=jax.ShapeDtypeStruct((n_rows, _NSEL), jnp.float32),
    )


_sc_kernel = _make_sc_kernel(_SC_ROWS)
_tc_kernel = _make_tc_kernel(_SC_ROWS, _ROWS - _SC_ROWS)


def kernel(inputs):
    x2d = inputs.reshape(_ROWS, _F)
    sc_out = _sc_kernel(x2d)
    tc_out = _tc_kernel(*([x2d] * _NSEL))
    out = jnp.concatenate([sc_out, tc_out], axis=0)
    return out.reshape(_B, _S, _NSEL)
